# baseline (speedup 1.0000x reference)
import functools

import jax
import jax.numpy as jnp
from jax import lax
from jax.experimental import pallas as pl
from jax.experimental.pallas import tpu as pltpu
from jax.experimental.pallas import tpu_sc as plsc

SCALE = 8.0


@functools.cache
def _build(B0, S, V, D):
    info = plsc.get_sparse_core_info()
    NC, NS, L = info.num_cores, info.num_subcores, info.num_lanes
    NW = NC * NS
    assert NW == 32 and L == 16 and D == 64 and S % 8 == 0 and B0 % 128 == 0
    n_blk = S // 8
    n_chunks = 4 * n_blk
    mesh = plsc.VectorSubcoreMesh(core_axis_name="c", subcore_axis_name="s")

    scratch = (
        [pltpu.VMEM((n_blk, 8, 128), jnp.int32)]
        + [pltpu.VMEM((256, D), jnp.float32) for _ in range(4)]
        + [pltpu.VMEM((16, 1, 8, 128), jnp.float32) for _ in range(2)]
        + [pltpu.SemaphoreType.DMA for _ in range(6)]
    )

    @functools.partial(
        pl.kernel,
        mesh=mesh,
        out_type=jax.ShapeDtypeStruct((S * D // 8, B0 // 128, 8, 128), jnp.float32),
        scratch_types=scratch,
        compiler_params=pltpu.CompilerParams(
            use_tc_tiling_on_sc=False, needs_layout_passes=False
        ),
    )
    def emb(idx_hbm, table_hbm, out_hbm, idx_v, *bufs):
        gbuf = list(bufs[:4])
        blk = list(bufs[4:6])
        gsem = list(bufs[6:10])
        osem = list(bufs[10:12])

        w = lax.axis_index("s") * NC + lax.axis_index("c")

        pltpu.sync_copy(idx_hbm.at[w], idx_v)

        iota = lax.iota(jnp.int32, L)
        rb_add = iota >> 3
        r_vec = iota & 7
        zero_vec = iota * 0

        def start_gathers(cc, gi):
            for s_loc in range(2):
                pltpu.async_copy(
                    table_hbm.at[idx_v.at[gi, 2 * cc + s_loc]],
                    gbuf[cc].at[pl.ds(128 * s_loc, 128)],
                    gsem[cc],
                )

        def out_slice(cc, gi):
            return out_hbm.at[pl.ds(64 * gi + 16 * cc, 16), pl.ds(w, 1)]

        for cc in range(4):
            start_gathers(cc, 0)

        def grp_body(gi, carry):
            for cc in range(4):
                ob = cc % 2
                for s_loc in range(2):
                    pltpu.make_async_copy(
                        table_hbm.at[idx_v.at[0, 0]],
                        gbuf[cc].at[pl.ds(128 * s_loc, 128)],
                        gsem[cc],
                    ).wait()
                if cc < 2:

                    @pl.when(gi >= 1)
                    def _():
                        pltpu.make_async_copy(
                            blk[ob], out_slice(cc, gi), osem[ob]
                        ).wait()

                else:
                    pltpu.make_async_copy(
                        blk[ob], out_slice(cc, gi), osem[ob]
                    ).wait()

                for s_loc in range(2):

                    def row_body(r_in, c, cc=cc, ob=ob, s_loc=s_loc):
                        c_vec = zero_vec + r_in
                        for j in range(D // L):
                            vec = gbuf[cc][128 * s_loc + r_in, pl.ds(L * j, L)]
                            plsc.store_scatter(
                                blk[ob],
                                [rb_add + (8 * s_loc + 2 * j), zero_vec, r_vec, c_vec],
                                vec * SCALE,
                            )
                        return c

                    lax.fori_loop(0, 128, row_body, 0)

                pltpu.async_copy(blk[ob], out_slice(cc, gi), osem[ob])

                @pl.when(gi < n_blk - 1)
                def _():
                    start_gathers(cc, gi + 1)

            return carry

        lax.fori_loop(0, n_blk, grp_body, 0)

        for cc in range(2, 4):
            pltpu.make_async_copy(
                blk[cc % 2], out_slice(cc, n_blk - 1), osem[cc % 2]
            ).wait()

    return emb


def kernel(inputs, table):
    B0, S = inputs.shape
    V, D = table.shape
    idx = (
        inputs.T.reshape(S // 8, 8, B0 // 128, 128)
        .transpose(2, 0, 1, 3)
        .astype(jnp.int32)
    )
    o4 = _build(B0, S, V, D)(idx, table)
    o = o4.transpose(0, 2, 1, 3).reshape(S * D, B0).reshape(S, D, B0)
    return o.transpose(2, 0, 1)


# speedup vs baseline: 1.0137x; 1.0137x over previous
import functools

import jax
import jax.numpy as jnp
from jax import lax
from jax.experimental import pallas as pl
from jax.experimental.pallas import tpu as pltpu
from jax.experimental.pallas import tpu_sc as plsc

SCALE = 8.0


@functools.cache
def _build(B0, S, V, D):
    info = plsc.get_sparse_core_info()
    NC, NS, L = info.num_cores, info.num_subcores, info.num_lanes
    NW = NC * NS
    assert NW == 32 and L == 16 and D == 64 and S % 8 == 0 and B0 % 128 == 0
    n_blk = S // 8
    n_chunks = 4 * n_blk
    mesh = plsc.VectorSubcoreMesh(core_axis_name="c", subcore_axis_name="s")

    scratch = (
        [pltpu.VMEM((n_blk, 8, 128), jnp.int32)]
        + [pltpu.VMEM((256, D), jnp.float32) for _ in range(4)]
        + [pltpu.VMEM((16, 1024), jnp.float32) for _ in range(2)]
        + [pltpu.SemaphoreType.DMA for _ in range(6)]
    )

    @functools.partial(
        pl.kernel,
        mesh=mesh,
        out_type=jax.ShapeDtypeStruct((S * D // 8, B0 * 8), jnp.float32),
        scratch_types=scratch,
        compiler_params=pltpu.CompilerParams(
            use_tc_tiling_on_sc=False, needs_layout_passes=False
        ),
    )
    def emb(idx_hbm, table_hbm, out_hbm, idx_v, *bufs):
        gbuf = list(bufs[:4])
        blk = list(bufs[4:6])
        gsem = list(bufs[6:10])
        osem = list(bufs[10:12])

        w = lax.axis_index("s") * NC + lax.axis_index("c")

        pltpu.sync_copy(idx_hbm.at[w], idx_v)

        iota = lax.iota(jnp.int32, L)
        rb_add = iota >> 3
        inner_base = (iota & 7) * 128
        rb_vecs = [[rb_add + (8 * s_loc + 2 * j) for j in range(D // L)]
                   for s_loc in range(2)]

        def start_gathers(cc, gi):
            for s_loc in range(2):
                pltpu.async_copy(
                    table_hbm.at[idx_v.at[gi, 2 * cc + s_loc]],
                    gbuf[cc].at[pl.ds(128 * s_loc, 128)],
                    gsem[cc],
                )

        def out_slice(cc, gi):
            return out_hbm.at[
                pl.ds(64 * gi + 16 * cc, 16), pl.ds(1024 * w, 1024)
            ]

        for cc in range(4):
            start_gathers(cc, 0)

        def grp_body(gi, carry):
            for cc in range(4):
                ob = cc % 2
                for s_loc in range(2):
                    pltpu.make_async_copy(
                        table_hbm.at[idx_v.at[0, 0]],
                        gbuf[cc].at[pl.ds(128 * s_loc, 128)],
                        gsem[cc],
                    ).wait()
                if cc < 2:

                    @pl.when(gi >= 1)
                    def _():
                        pltpu.make_async_copy(
                            blk[ob], out_slice(cc, gi), osem[ob]
                        ).wait()

                else:
                    pltpu.make_async_copy(
                        blk[ob], out_slice(cc, gi), osem[ob]
                    ).wait()

                def row_body(r2, c, cc=cc, ob=ob):
                    for rr in range(2):
                        r_in = r2 * 2 + rr
                        inner = inner_base + r_in
                        for s_loc in range(2):
                            for j in range(D // L):
                                vec = gbuf[cc][
                                    128 * s_loc + r_in, pl.ds(L * j, L)
                                ]
                                plsc.store_scatter(
                                    blk[ob],
                                    [rb_vecs[s_loc][j], inner],
                                    vec * SCALE,
                                )
                    return c

                lax.fori_loop(0, 64, row_body, 0)

                pltpu.async_copy(blk[ob], out_slice(cc, gi), osem[ob])

                @pl.when(gi < n_blk - 1)
                def _():
                    start_gathers(cc, gi + 1)

            return carry

        lax.fori_loop(0, n_blk, grp_body, 0)

        for cc in range(2, 4):
            pltpu.make_async_copy(
                blk[cc % 2], out_slice(cc, n_blk - 1), osem[cc % 2]
            ).wait()

    return emb


def kernel(inputs, table):
    B0, S = inputs.shape
    V, D = table.shape
    idx = (
        inputs.T.reshape(S // 8, 8, B0 // 128, 128)
        .transpose(2, 0, 1, 3)
        .astype(jnp.int32)
    )
    o2 = _build(B0, S, V, D)(idx, table)
    o4 = o2.reshape(S * D // 8, B0 // 128, 8, 128)
    o = o4.transpose(0, 2, 1, 3).reshape(S * D, B0).reshape(S, D, B0)
    return o.transpose(2, 0, 1)


# speedup vs baseline: 1.5576x; 1.5365x over previous
import functools

import jax
import jax.numpy as jnp
from jax import lax
from jax.experimental import pallas as pl
from jax.experimental.pallas import tpu as pltpu
from jax.experimental.pallas import tpu_sc as plsc

SCALE = 8.0
C = 100


@functools.cache
def _build(B0, S, V, D):
    info = plsc.get_sparse_core_info()
    NC, NS, L = info.num_cores, info.num_subcores, info.num_lanes
    NW = NC * NS
    assert S == 2 * C and B0 % (2 * NW) == 0
    rows_per_w = B0 // NW
    n_chunks = 2 * rows_per_w
    n_grp = rows_per_w // 2

    mesh = plsc.VectorSubcoreMesh(core_axis_name="c", subcore_axis_name="s")

    scratch = (
        [pltpu.VMEM((n_chunks, C), jnp.int32)]
        + [pltpu.VMEM((C, 2 * D), jnp.float32) for _ in range(2)]
        + [pltpu.VMEM((S, D), jnp.float32) for _ in range(2)]
        + [pltpu.SemaphoreType.DMA for _ in range(4)]
    )

    @functools.partial(
        pl.kernel,
        mesh=mesh,
        out_type=jax.ShapeDtypeStruct((B0, S, D), jnp.float32),
        scratch_types=scratch,
    )
    def emb(idx_hbm, table_hbm, out_hbm, idx_v, g0, g1, o0, o1, gs0, gs1, os0, os1):
        rows_g = [g0, g1]
        rows_o = [o0, o1]
        gsem = [gs0, gs1]
        osem = [os0, os1]

        wid = lax.axis_index("s") * NC + lax.axis_index("c")
        crow0 = wid * n_chunks
        orow0 = wid * rows_per_w

        pltpu.sync_copy(idx_hbm.at[pl.ds(crow0, n_chunks)], idx_v)

        def start_gather(h, c):
            pltpu.async_copy(table_hbm.at[idx_v.at[c]], rows_g[h], gsem[h])

        start_gather(0, 0)
        start_gather(1, 1)

        def grp_body(gi, carry):
            for lq in range(2):
                q = gi * 2 + lq
                @pl.when(gi >= 1)
                def _():
                    pltpu.make_async_copy(
                        rows_o[lq], out_hbm.at[orow0 + q], osem[lq]
                    ).wait()

                for h in range(2):
                    pltpu.make_async_copy(
                        table_hbm.at[idx_v.at[0]], rows_g[h], gsem[h]
                    ).wait()

                    def scale_body(r4, c, h=h, lq=lq):
                        for rr in range(4):
                            r = r4 * 4 + rr
                            for j in range(D // L):
                                rows_o[lq][h * C + r, pl.ds(j * L, L)] = (
                                    rows_g[h][r, pl.ds(j * L, L)] * SCALE
                                )
                        return c

                    lax.fori_loop(0, C // 4, scale_body, 0)

                    @pl.when(q < rows_per_w - 1)
                    def _():
                        start_gather(h, (q + 1) * 2 + h)

                pltpu.async_copy(rows_o[lq], out_hbm.at[orow0 + q], osem[lq])

            return carry

        lax.fori_loop(0, n_grp, grp_body, 0)

        for lq in range(2):
            q = rows_per_w - 2 + lq
            pltpu.make_async_copy(
                rows_o[lq], out_hbm.at[orow0 + q], osem[lq]
            ).wait()

    return emb


def kernel(inputs, table):
    B0, S = inputs.shape
    V, D = table.shape
    idx = inputs.reshape(B0 * S // C, C).astype(jnp.int32)
    t128 = jnp.pad(table, ((0, 0), (0, D)))
    return _build(B0, S, V, D)(idx, t128)


# speedup vs baseline: 1.5619x; 1.0027x over previous
import functools

import jax
import jax.numpy as jnp
from jax import lax
from jax.experimental import pallas as pl
from jax.experimental.pallas import tpu as pltpu
from jax.experimental.pallas import tpu_sc as plsc

SCALE = 8.0
C = 100


@functools.cache
def _build(B0, S, V, D):
    info = plsc.get_sparse_core_info()
    NC, NS, L = info.num_cores, info.num_subcores, info.num_lanes
    NW = NC * NS
    assert S == 2 * C and B0 % (2 * NW) == 0
    rows_per_w = B0 // NW
    n_chunks = 2 * rows_per_w
    n_grp = rows_per_w // 2

    mesh = plsc.VectorSubcoreMesh(core_axis_name="c", subcore_axis_name="s")

    n_q = n_chunks // 64
    scratch = (
        [pltpu.VMEM((64, C), jnp.int32) for _ in range(2)]
        + [pltpu.VMEM((C, 2 * D), jnp.float32) for _ in range(4)]
        + [pltpu.VMEM((S, D), jnp.float32) for _ in range(2)]
        + [pltpu.SemaphoreType.DMA for _ in range(6)]
    )

    @functools.partial(
        pl.kernel,
        mesh=mesh,
        out_type=jax.ShapeDtypeStruct((B0, S, D), jnp.float32),
        scratch_types=scratch,
    )
    def emb(
        idx_hbm, table_hbm, out_hbm, iq0, iq1, g0, g1, g2, g3,
        o0, o1, gs0, gs1, gs2, gs3, os0, os1,
    ):
        idxq = [iq0, iq1]
        rows_g = [g0, g1, g2, g3]
        rows_o = [o0, o1]
        gsem = [gs0, gs1, gs2, gs3]
        osem = [os0, os1]

        wid = lax.axis_index("s") * NC + lax.axis_index("c")
        crow0 = wid * n_chunks
        orow0 = wid * rows_per_w

        def stage_quarter(k):
            @pl.when(k % 2 == 0)
            def _():
                pltpu.sync_copy(idx_hbm.at[pl.ds(crow0 + 64 * k, 64)], idxq[0])

            @pl.when(k % 2 == 1)
            def _():
                pltpu.sync_copy(idx_hbm.at[pl.ds(crow0 + 64 * k, 64)], idxq[1])

        stage_quarter(0)

        def start_gather(b, c):
            @pl.when((c // 64) % 2 == 0)
            def _():
                pltpu.async_copy(
                    table_hbm.at[idxq[0].at[c % 64]], rows_g[b], gsem[b]
                )

            @pl.when((c // 64) % 2 == 1)
            def _():
                pltpu.async_copy(
                    table_hbm.at[idxq[1].at[c % 64]], rows_g[b], gsem[b]
                )

        for b in range(4):
            start_gather(b, b)

        def grp_body(gi, carry):
            @pl.when(((gi + 2) % 16 == 0) & (gi + 2 < 16 * n_q))
            def _():
                stage_quarter((gi + 2) // 16)

            for lq in range(2):
                q = gi * 2 + lq
                @pl.when(gi >= 1)
                def _():
                    pltpu.make_async_copy(
                        rows_o[lq], out_hbm.at[orow0 + q], osem[lq]
                    ).wait()

                for h in range(2):
                    b = 2 * lq + h
                    pltpu.make_async_copy(
                        table_hbm.at[idxq[0].at[0]], rows_g[b], gsem[b]
                    ).wait()

                    def scale_body(r4, c, b=b, h=h, lq=lq):
                        for rr in range(4):
                            r = r4 * 4 + rr
                            for j in range(D // L):
                                rows_o[lq][h * C + r, pl.ds(j * L, L)] = (
                                    rows_g[b][r, pl.ds(j * L, L)] * SCALE
                                )
                        return c

                    lax.fori_loop(0, C // 4, scale_body, 0)

                    @pl.when(gi < n_grp - 1)
                    def _():
                        start_gather(b, 4 * (gi + 1) + b)

                pltpu.async_copy(rows_o[lq], out_hbm.at[orow0 + q], osem[lq])

            return carry

        lax.fori_loop(0, n_grp, grp_body, 0)

        for lq in range(2):
            q = rows_per_w - 2 + lq
            pltpu.make_async_copy(
                rows_o[lq], out_hbm.at[orow0 + q], osem[lq]
            ).wait()

    return emb


def kernel(inputs, table):
    B0, S = inputs.shape
    V, D = table.shape
    idx = inputs.reshape(B0 * S // C, C).astype(jnp.int32)
    t128 = jnp.pad(table, ((0, 0), (0, D)))
    return _build(B0, S, V, D)(idx, t128)
